# baseline (speedup 1.0000x reference)
import dataclasses
import functools

import jax
import jax.numpy as jnp
from jax import lax
from jax.experimental import pallas as pl
from jax.experimental.pallas import tpu as pltpu
from jax.experimental.pallas import tpu_sc as plsc

_N = 10000
_D = 128
_E = 320000

_NC = 2
_NS = 16
_NW = _NC * _NS

_CHUNK = 80
_NCHUNKS = _E // _CHUNK
_AN = 10240
_RPT = _AN // _NS

_mesh = plsc.VectorSubcoreMesh(core_axis_name="c", subcore_axis_name="s")


def _segsum_body(x_hbm, src_hbm, dst_hbm, out_hbm, acc, srcv, dstv,
                 rows0, rows1, sg0, sg1, ss0, ss1, si0, si1, si2):
    core = lax.axis_index("c")
    sid = lax.axis_index("s")
    wid = core * _NS + sid

    rows_bufs = (rows0, rows1)
    gsems = (sg0, sg1)
    ssems = (ss0, ss1)
    isems = (si0, si1, si2)

    @pl.loop(0, _CHUNK)
    def _(i):
        @pl.loop(0, _D // 16)
        def _(j):
            rows0[i, pl.ds(j * 16, 16)] = jnp.zeros((16,), jnp.float32)

    row0 = sid * _RPT

    @pl.loop(0, _RPT // _CHUNK)
    def _(i):
        pltpu.sync_copy(rows0.at[pl.ds(0, _CHUNK)],
                        acc.at[pl.ds(row0 + i * _CHUNK, _CHUNK)])

    plsc.subcore_barrier()

    def start_idx(c, k):
        base = c * _CHUNK
        pltpu.async_copy(src_hbm.at[pl.ds(base, _CHUNK)], srcv.at[k], isems[k])
        pltpu.async_copy(dst_hbm.at[pl.ds(base, _CHUNK)], dstv.at[k], isems[k])

    def wait_idx(c, k):
        base = c * _CHUNK
        pltpu.make_async_copy(src_hbm.at[pl.ds(base, _CHUNK)], srcv.at[k],
                              isems[k]).wait()
        pltpu.make_async_copy(dst_hbm.at[pl.ds(base, _CHUNK)], dstv.at[k],
                              isems[k]).wait()

    def start_gather(b, k):
        pltpu.async_copy(x_hbm.at[srcv.at[k]], rows_bufs[b], gsems[b])

    def wait_gather(b, k):
        pltpu.make_async_copy(x_hbm.at[srcv.at[k]], rows_bufs[b],
                              gsems[b]).wait()

    def start_scatter(b, k):
        pltpu.async_copy(rows_bufs[b], acc.at[dstv.at[k]], ssems[b],
                         add=True)

    def wait_scatter(b, k):
        pltpu.make_async_copy(rows_bufs[b], acc.at[dstv.at[k]],
                              ssems[b]).wait()

    start_idx(wid, 0)
    start_idx(wid + _NW, 1)
    wait_idx(wid, 0)
    start_gather(0, 0)

    @pl.loop(0, _NCHUNKS // _NW + 1, step=6)
    def _(i):
        for t in range(6):
            b = t & 1
            k = t % 3
            k1 = (k + 1) % 3
            k2 = (k + 2) % 3
            c0 = wid + (i + t) * _NW

            @pl.when(c0 < _NCHUNKS)
            def _():
                wait_gather(b, k)
                c1 = c0 + _NW
                c2 = c0 + 2 * _NW

                @pl.when(c1 < _NCHUNKS)
                def _():
                    wait_idx(c1, k1)

                    @pl.when(c0 >= _NW)
                    def _():
                        wait_scatter(1 - b, k2)

                    start_gather(1 - b, k1)

                start_scatter(b, k)

                @pl.when(c2 < _NCHUNKS)
                def _():
                    start_idx(c2, k2)

    wait_scatter(1, 0)
    wait_scatter(0, 1)

    plsc.subcore_barrier()

    pltpu.sync_copy(acc.at[pl.ds(row0, _RPT)],
                    out_hbm.at[core, pl.ds(row0, _RPT)])


_segsum = pl.kernel(
    _segsum_body,
    mesh=_mesh,
    out_type=[jax.ShapeDtypeStruct((_NC, _AN, _D), jnp.float32)],
    scratch_types=[
        pltpu.VMEM_SHARED((_AN, _D), jnp.float32),
        pltpu.VMEM((3, _CHUNK), jnp.int32),
        pltpu.VMEM((3, _CHUNK), jnp.int32),
        pltpu.VMEM((_CHUNK, _D), jnp.float32),
        pltpu.VMEM((_CHUNK, _D), jnp.float32),
        pltpu.SemaphoreType.DMA,
        pltpu.SemaphoreType.DMA,
        pltpu.SemaphoreType.DMA,
        pltpu.SemaphoreType.DMA,
        pltpu.SemaphoreType.DMA,
        pltpu.SemaphoreType.DMA,
        pltpu.SemaphoreType.DMA,
    ],
)


def _counts_body(dst_hbm, cnt_hbm, cnt8, dstv, si0, si1):
    core = lax.axis_index("c")
    sid = lax.axis_index("s")
    wid = core * _NS + sid

    @pl.loop(0, 8 * _AN // 16)
    def _(i):
        cnt8[pl.ds(i * 16, 16)] = jnp.zeros((16,), jnp.float32)

    lanes = lax.iota(jnp.int32, 16)
    sub_off = (lanes & 7) * _AN
    m_lo = lanes < 8
    m_hi = lanes >= 8
    ones16 = jnp.full((16,), 1.0, jnp.float32)
    isems = (si0, si1)

    def start_idx(c, b):
        pltpu.async_copy(dst_hbm.at[pl.ds(c * _CHUNK, _CHUNK)], dstv.at[b],
                         isems[b])

    def wait_idx(c, b):
        pltpu.make_async_copy(dst_hbm.at[pl.ds(c * _CHUNK, _CHUNK)],
                              dstv.at[b], isems[b]).wait()

    start_idx(wid, 0)
    start_idx(wid + _NW, 1)

    @pl.loop(0, _NCHUNKS // _NW + 2, step=2)
    def _(i):
        for b in (0, 1):
            c0 = wid + (i + b) * _NW

            @pl.when(c0 < _NCHUNKS)
            def _():
                wait_idx(c0, b)
                c2 = c0 + 2 * _NW

                @pl.loop(0, _CHUNK // 16)
                def _(g):
                    d = dstv[b, pl.ds(g * 16, 16)]
                    idx = sub_off + d
                    plsc.addupdate_scatter(cnt8, [idx], ones16, mask=m_lo)
                    plsc.addupdate_scatter(cnt8, [idx], ones16, mask=m_hi)

                @pl.when(c2 < _NCHUNKS)
                def _():
                    start_idx(c2, b)

    @pl.loop(0, _AN // 16)
    def _(j):
        t = cnt8[pl.ds(j * 16, 16)]
        for s in range(1, 8):
            t = t + cnt8[pl.ds(s * _AN + j * 16, 16)]
        cnt8[pl.ds(j * 16, 16)] = t

    pltpu.sync_copy(cnt8.at[pl.ds(0, _AN)], cnt_hbm.at[wid])


_cp = pltpu.CompilerParams()
if "needs_layout_passes" in pltpu.CompilerParams.__dataclass_fields__:
    _cp = dataclasses.replace(_cp, needs_layout_passes=False)

_counts = pl.kernel(
    _counts_body,
    mesh=_mesh,
    compiler_params=_cp,
    out_type=[jax.ShapeDtypeStruct((_NW, _AN), jnp.float32)],
    scratch_types=[
        pltpu.VMEM((8 * _AN,), jnp.float32),
        pltpu.VMEM((2, _CHUNK), jnp.int32),
        pltpu.SemaphoreType.DMA,
        pltpu.SemaphoreType.DMA,
    ],
)


_BN = 512


def _tc_layer_body(relu, p_ref, c_ref, x_ref, wl_ref, b_ref, wr_ref, o_ref):
    s = p_ref[0] + p_ref[1]
    pid = pl.program_id(0)
    cnt = jnp.sum(c_ref[:, pl.ds(pid * _BN, _BN)], axis=0)[:, None]
    del pid
    mean = s / jnp.maximum(cnt, 1.0)
    acc = lax.dot_general(mean, wl_ref[...], (((1,), (1,)), ((), ())),
                          preferred_element_type=jnp.float32)
    acc = acc + lax.dot_general(x_ref[...], wr_ref[...], (((1,), (1,)), ((), ())),
                                preferred_element_type=jnp.float32)
    acc = acc + b_ref[...]
    o_ref[...] = jnp.maximum(acc, 0.0) if relu else acc


def _tc_layer(partials, cnts, x, Wl, b, Wr, relu):
    return pl.pallas_call(
        functools.partial(_tc_layer_body, relu),
        grid=(_AN // _BN,),
        in_specs=[
            pl.BlockSpec((_NC, _BN, _D), lambda i: (0, i, 0)),
            pl.BlockSpec((_NW, _AN), lambda i: (0, 0)),
            pl.BlockSpec((_BN, _D), lambda i: (i, 0)),
            pl.BlockSpec((_D, _D), lambda i: (0, 0)),
            pl.BlockSpec((1, _D), lambda i: (0, 0)),
            pl.BlockSpec((_D, _D), lambda i: (0, 0)),
        ],
        out_specs=pl.BlockSpec((_BN, _D), lambda i: (i, 0)),
        out_shape=jax.ShapeDtypeStruct((_AN, _D), jnp.float32),
    )(partials, cnts, x, Wl, b.reshape(1, _D), Wr)


def kernel(x, edge_index, W1l, b1, W1r, W2l, b2, W2r):
    src = edge_index[0]
    dst = edge_index[1]
    x_p = jnp.concatenate([x, jnp.zeros((_AN - _N, _D), jnp.float32)])
    c1, = _counts(dst)
    p1, = _segsum(x_p, src, dst)
    h = _tc_layer(p1, c1, x_p, W1l, b1, W1r, relu=True)
    p2, = _segsum(h, src, dst)
    out = _tc_layer(p2, c1, h, W2l, b2, W2r, relu=False)
    return out[:_N]


# speedup vs baseline: 1.0406x; 1.0406x over previous
import dataclasses
import functools

import jax
import jax.numpy as jnp
from jax import lax
from jax.experimental import pallas as pl
from jax.experimental.pallas import tpu as pltpu
from jax.experimental.pallas import tpu_sc as plsc

_N = 10000
_D = 128
_E = 320000

_NC = 2
_NS = 16
_NW = _NC * _NS

_CHUNK = 80
_NCHUNKS = _E // _CHUNK
_AN = 10240
_RPT = _AN // _NS

_mesh = plsc.VectorSubcoreMesh(core_axis_name="c", subcore_axis_name="s")


def _segsum_body(x_hbm, src_hbm, dst_hbm, out_hbm, acc, srcv, dstv,
                 rows0, rows1, sg0, sg1, ss0, ss1, si0, si1, si2):
    core = lax.axis_index("c")
    sid = lax.axis_index("s")
    wid = core * _NS + sid

    rows_bufs = (rows0, rows1)
    gsems = (sg0, sg1)
    ssems = (ss0, ss1)
    isems = (si0, si1, si2)

    @pl.loop(0, _CHUNK)
    def _(i):
        @pl.loop(0, _D // 16)
        def _(j):
            rows0[i, pl.ds(j * 16, 16)] = jnp.zeros((16,), jnp.float32)

    row0 = sid * _RPT

    @pl.loop(0, _RPT // _CHUNK)
    def _(i):
        pltpu.sync_copy(rows0.at[pl.ds(0, _CHUNK)],
                        acc.at[pl.ds(row0 + i * _CHUNK, _CHUNK)])

    plsc.subcore_barrier()

    def start_idx(c, k):
        base = c * _CHUNK
        pltpu.async_copy(src_hbm.at[pl.ds(base, _CHUNK)], srcv.at[k], isems[k])
        pltpu.async_copy(dst_hbm.at[pl.ds(base, _CHUNK)], dstv.at[k], isems[k])

    def wait_idx(c, k):
        base = c * _CHUNK
        pltpu.make_async_copy(src_hbm.at[pl.ds(base, _CHUNK)], srcv.at[k],
                              isems[k]).wait()
        pltpu.make_async_copy(dst_hbm.at[pl.ds(base, _CHUNK)], dstv.at[k],
                              isems[k]).wait()

    def start_gather(b, k):
        pltpu.async_copy(x_hbm.at[srcv.at[k]], rows_bufs[b], gsems[b])

    def wait_gather(b, k):
        pltpu.make_async_copy(x_hbm.at[srcv.at[k]], rows_bufs[b],
                              gsems[b]).wait()

    def start_scatter(b, k):
        pltpu.async_copy(rows_bufs[b], acc.at[dstv.at[k]], ssems[b],
                         add=True)

    def wait_scatter(b, k):
        pltpu.make_async_copy(rows_bufs[b], acc.at[dstv.at[k]],
                              ssems[b]).wait()

    start_idx(wid, 0)
    start_idx(wid + _NW, 1)
    wait_idx(wid, 0)
    start_gather(0, 0)

    @pl.loop(0, _NCHUNKS // _NW + 1, step=6)
    def _(i):
        for t in range(6):
            b = t & 1
            k = t % 3
            k1 = (k + 1) % 3
            k2 = (k + 2) % 3
            c0 = wid + (i + t) * _NW

            @pl.when(c0 < _NCHUNKS)
            def _():
                wait_gather(b, k)
                c1 = c0 + _NW
                c2 = c0 + 2 * _NW

                @pl.when(c1 < _NCHUNKS)
                def _():
                    wait_idx(c1, k1)

                    @pl.when(c0 >= _NW)
                    def _():
                        wait_scatter(1 - b, k2)

                    start_gather(1 - b, k1)

                start_scatter(b, k)

                @pl.when(c2 < _NCHUNKS)
                def _():
                    start_idx(c2, k2)

    wait_scatter(1, 0)
    wait_scatter(0, 1)

    plsc.subcore_barrier()

    pltpu.sync_copy(acc.at[pl.ds(row0, _RPT)],
                    out_hbm.at[core, pl.ds(row0, _RPT)])


_segsum = pl.kernel(
    _segsum_body,
    mesh=_mesh,
    out_type=[jax.ShapeDtypeStruct((_NC, _AN, _D), jnp.float32)],
    scratch_types=[
        pltpu.VMEM_SHARED((_AN, _D), jnp.float32),
        pltpu.VMEM((3, _CHUNK), jnp.int32),
        pltpu.VMEM((3, _CHUNK), jnp.int32),
        pltpu.VMEM((_CHUNK, _D), jnp.float32),
        pltpu.VMEM((_CHUNK, _D), jnp.float32),
        pltpu.SemaphoreType.DMA,
        pltpu.SemaphoreType.DMA,
        pltpu.SemaphoreType.DMA,
        pltpu.SemaphoreType.DMA,
        pltpu.SemaphoreType.DMA,
        pltpu.SemaphoreType.DMA,
        pltpu.SemaphoreType.DMA,
    ],
)


def _counts_body(dst_hbm, cnt_hbm, cnt8, dstv, si0, si1, si2, si3, si4, si5):
    core = lax.axis_index("c")
    sid = lax.axis_index("s")
    wid = core * _NS + sid

    @pl.loop(0, 8 * _AN // 16)
    def _(i):
        cnt8[pl.ds(i * 16, 16)] = jnp.zeros((16,), jnp.float32)

    lanes = lax.iota(jnp.int32, 16)
    sub_off = (lanes & 7) * _AN
    m_lo = lanes < 8
    m_hi = lanes >= 8
    ones16 = jnp.full((16,), 1.0, jnp.float32)
    isems = (si0, si1, si2, si3, si4, si5)

    def start_idx(c, b):
        pltpu.async_copy(dst_hbm.at[pl.ds(c * _CHUNK, _CHUNK)], dstv.at[b],
                         isems[b])

    def wait_idx(c, b):
        pltpu.make_async_copy(dst_hbm.at[pl.ds(c * _CHUNK, _CHUNK)],
                              dstv.at[b], isems[b]).wait()

    for p in range(5):
        start_idx(wid + p * _NW, p)

    @pl.loop(0, _NCHUNKS // _NW + 1, step=6)
    def _(i):
        for t in range(6):
            b = t % 6
            c0 = wid + (i + t) * _NW

            @pl.when(c0 < _NCHUNKS)
            def _():
                wait_idx(c0, b)
                c5 = c0 + 5 * _NW

                @pl.loop(0, _CHUNK // 16)
                def _(g):
                    d = dstv[b, pl.ds(g * 16, 16)]
                    idx = sub_off + d
                    plsc.addupdate_scatter(cnt8, [idx], ones16, mask=m_lo)
                    plsc.addupdate_scatter(cnt8, [idx], ones16, mask=m_hi)

                @pl.when(c5 < _NCHUNKS)
                def _():
                    start_idx(c5, (t + 5) % 6)

    @pl.loop(0, _AN // 16)
    def _(j):
        t = cnt8[pl.ds(j * 16, 16)]
        for s in range(1, 8):
            t = t + cnt8[pl.ds(s * _AN + j * 16, 16)]
        cnt8[pl.ds(j * 16, 16)] = t

    pltpu.sync_copy(cnt8.at[pl.ds(0, _AN)], cnt_hbm.at[wid])


_cp = pltpu.CompilerParams()
if "needs_layout_passes" in pltpu.CompilerParams.__dataclass_fields__:
    _cp = dataclasses.replace(_cp, needs_layout_passes=False)

_counts = pl.kernel(
    _counts_body,
    mesh=_mesh,
    compiler_params=_cp,
    out_type=[jax.ShapeDtypeStruct((_NW, _AN), jnp.float32)],
    scratch_types=[
        pltpu.VMEM((8 * _AN,), jnp.float32),
        pltpu.VMEM((6, _CHUNK), jnp.int32),
        pltpu.SemaphoreType.DMA,
        pltpu.SemaphoreType.DMA,
        pltpu.SemaphoreType.DMA,
        pltpu.SemaphoreType.DMA,
        pltpu.SemaphoreType.DMA,
        pltpu.SemaphoreType.DMA,
    ],
)


_BN = 512


def _tc_layer_body(relu, p_ref, c_ref, x_ref, wl_ref, b_ref, wr_ref, o_ref):
    s = p_ref[0] + p_ref[1]
    pid = pl.program_id(0)
    cnt = jnp.sum(c_ref[:, pl.ds(pid * _BN, _BN)], axis=0)[:, None]
    del pid
    mean = s / jnp.maximum(cnt, 1.0)
    acc = lax.dot_general(mean, wl_ref[...], (((1,), (1,)), ((), ())),
                          preferred_element_type=jnp.float32)
    acc = acc + lax.dot_general(x_ref[...], wr_ref[...], (((1,), (1,)), ((), ())),
                                preferred_element_type=jnp.float32)
    acc = acc + b_ref[...]
    o_ref[...] = jnp.maximum(acc, 0.0) if relu else acc


def _tc_layer(partials, cnts, x, Wl, b, Wr, relu):
    return pl.pallas_call(
        functools.partial(_tc_layer_body, relu),
        grid=(_AN // _BN,),
        in_specs=[
            pl.BlockSpec((_NC, _BN, _D), lambda i: (0, i, 0)),
            pl.BlockSpec((_NW, _AN), lambda i: (0, 0)),
            pl.BlockSpec((_BN, _D), lambda i: (i, 0)),
            pl.BlockSpec((_D, _D), lambda i: (0, 0)),
            pl.BlockSpec((1, _D), lambda i: (0, 0)),
            pl.BlockSpec((_D, _D), lambda i: (0, 0)),
        ],
        out_specs=pl.BlockSpec((_BN, _D), lambda i: (i, 0)),
        out_shape=jax.ShapeDtypeStruct((_AN, _D), jnp.float32),
    )(partials, cnts, x, Wl, b.reshape(1, _D), Wr)


def kernel(x, edge_index, W1l, b1, W1r, W2l, b2, W2r):
    src = edge_index[0]
    dst = edge_index[1]
    x_p = jnp.concatenate([x, jnp.zeros((_AN - _N, _D), jnp.float32)])
    c1, = _counts(dst)
    p1, = _segsum(x_p, src, dst)
    h = _tc_layer(p1, c1, x_p, W1l, b1, W1r, relu=True)
    p2, = _segsum(h, src, dst)
    out = _tc_layer(p2, c1, h, W2l, b2, W2r, relu=False)
    return out[:_N]
